# baseline (speedup 1.0000x reference)
import jax
import jax.numpy as jnp
from jax.experimental import pallas as pl
from jax.experimental.pallas import tpu as pltpu

NUM_EXPERTS = 16
TOP_K = 8
N_EMBD = 768
D_FFN = 384
BLK_T = 256


def _moe_kernel(x_ref, rw_ref, w1_ref, w2_ref, out_ref):
    xb = x_ref[...]
    logits = jax.lax.dot_general(
        xb, rw_ref[...], (((1,), (1,)), ((), ())),
        preferred_element_type=jnp.float32)
    col = jax.lax.broadcasted_iota(jnp.int32, logits.shape, 1)
    rank = jnp.zeros(logits.shape, dtype=jnp.int32)
    for j in range(NUM_EXPERTS):
        lj = logits[:, j:j + 1]
        beats = (lj > logits) | ((lj == logits) & (col > j))
        rank = rank + beats.astype(jnp.int32)
    sel = rank < TOP_K
    m = jnp.max(logits, axis=-1, keepdims=True)
    ew = jnp.where(sel, jnp.exp(logits - m), 0.0)
    comb = ew / jnp.sum(ew, axis=-1, keepdims=True)

    h = jax.lax.dot_general(
        xb.astype(jnp.bfloat16), w1_ref[...], (((1,), (0,)), ((), ())),
        preferred_element_type=jnp.float32)
    h = jax.nn.gelu(h)
    g = jnp.concatenate(
        [h[:, e * D_FFN:(e + 1) * D_FFN] * comb[:, e:e + 1]
         for e in range(NUM_EXPERTS)], axis=1)
    out_ref[...] = jax.lax.dot_general(
        g.astype(jnp.bfloat16), w2_ref[...], (((1,), (0,)), ((), ())),
        preferred_element_type=jnp.float32)


def kernel(x, router_w, w1, w2):
    B, S, D = x.shape
    T = B * S
    xt = x.reshape(T, D)
    EF = NUM_EXPERTS * D_FFN
    out = pl.pallas_call(
        _moe_kernel,
        grid=(T // BLK_T,),
        in_specs=[
            pl.BlockSpec((BLK_T, D), lambda t: (t, 0)),
            pl.BlockSpec((NUM_EXPERTS, D), lambda t: (0, 0)),
            pl.BlockSpec((D, EF), lambda t: (0, 0)),
            pl.BlockSpec((EF, D), lambda t: (0, 0)),
        ],
        out_specs=pl.BlockSpec((BLK_T, D), lambda t: (t, 0)),
        out_shape=jax.ShapeDtypeStruct((T, D), jnp.float32),
        compiler_params=pltpu.CompilerParams(
            dimension_semantics=("arbitrary",),
        ),
    )(xt, router_w, w1.astype(jnp.bfloat16), w2.astype(jnp.bfloat16))
    return out.reshape(B, S, D)
